# Initial kernel scaffold; baseline (speedup 1.0000x reference)
import jax
import jax.numpy as jnp
from jax.experimental import pallas as pl


def kernel(x, edge_index, in_W, in_b, W0, b0, W1, b1, W2, b2, W3, b3, ln_g, ln_b, oW1, ob1, oW2, ob2):
    raise NotImplementedError("write your pallas kernel here")


# speedup vs baseline: 14.1812x; 14.1812x over previous
import functools

import jax
import jax.numpy as jnp
from jax import lax
from jax.experimental import pallas as pl
from jax.experimental.pallas import tpu as pltpu
from jax.experimental.pallas import tpu_sc as plsc

_N = 10000
_D = 128
_E = 320000
_NC = 2
_NS = 16
_NW = _NC * _NS
_EPT = _E // _NW
_CH = 80
_NCH = _EPT // _CH
_NPAD = 10240
_RPT = _NPAD // _NS
_DW = 128

_mesh = plsc.VectorSubcoreMesh(core_axis_name="c", subcore_axis_name="s")


@functools.partial(
    pl.kernel,
    out_type=jax.ShapeDtypeStruct((_NC, _NPAD, _DW), jnp.float32),
    mesh=_mesh,
    scratch_types=[
        pltpu.VMEM((_NCH, _CH), jnp.int32),
        pltpu.VMEM((_CH, _DW), jnp.float32),
        pltpu.VMEM_SHARED((_NPAD, _DW), jnp.float32),
    ],
)
def _sc_degree(dst_hbm, ones_hbm, zrow_hbm, out_hbm, dst_v, ones_v, acc):
    c = lax.axis_index("c")
    s = lax.axis_index("s")
    wid = c * _NS + s
    pltpu.sync_copy(zrow_hbm, acc.at[pl.ds(s * _RPT, _RPT)])
    pltpu.sync_copy(ones_hbm, ones_v)
    pltpu.sync_copy(dst_hbm.at[wid], dst_v)
    plsc.subcore_barrier()

    def body(j, carry):
        pltpu.sync_copy(ones_v, acc.at[dst_v.at[j]], add=True)
        return carry

    lax.fori_loop(0, _NCH, body, 0)
    plsc.subcore_barrier()
    pltpu.sync_copy(acc.at[pl.ds(s * _RPT, _RPT)],
                    out_hbm.at[c, pl.ds(s * _RPT, _RPT)])


@functools.partial(
    pl.kernel,
    out_type=jax.ShapeDtypeStruct((_NC, _NPAD, _D), jnp.float32),
    mesh=_mesh,
    scratch_types=[
        pltpu.VMEM((_NCH, _CH), jnp.int32),
        pltpu.VMEM((_NCH, _CH), jnp.int32),
        pltpu.VMEM((_CH, _D), jnp.float32),
        pltpu.VMEM((_CH, _D), jnp.float32),
        pltpu.VMEM_SHARED((_NPAD, _D), jnp.float32),
        pltpu.SemaphoreType.DMA,
    ],
)
def _sc_scatter(g_hbm, src_hbm, dst_hbm, zeros_hbm, out_hbm,
                src_v, dst_v, buf0, buf1, acc, sem0):
    sem1 = sem0
    c = lax.axis_index("c")
    s = lax.axis_index("s")
    wid = c * _NS + s
    pltpu.sync_copy(zeros_hbm, acc.at[pl.ds(s * _RPT, _RPT)])
    pltpu.sync_copy(src_hbm.at[wid], src_v)
    pltpu.sync_copy(dst_hbm.at[wid], dst_v)
    plsc.subcore_barrier()

    @plsc.parallel_loop(0, _NCH, step=1, unroll=2)
    def body(j):
        pltpu.async_copy(g_hbm.at[src_v.at[j]], buf0, sem0).wait()
        pltpu.sync_copy(buf0, acc.at[dst_v.at[j]], add=True)

    plsc.subcore_barrier()
    pltpu.sync_copy(acc.at[pl.ds(s * _RPT, _RPT)],
                    out_hbm.at[c, pl.ds(s * _RPT, _RPT)])


_BN = 2000


def _dinv_of(degp):
    deg = degp[0, :, 0] + degp[1, :, 0] + 1.0
    return lax.rsqrt(deg)[:, None]


def _tc_in_body(x, degp, in_W, in_b, W0, h_out, g_out):
    h = jnp.maximum(
        jnp.dot(x[...], in_W[...], preferred_element_type=jnp.float32)
        + in_b[...], 0.0)
    h_out[...] = h
    g_out[...] = jnp.dot(h, W0[...], preferred_element_type=jnp.float32) \
        * _dinv_of(degp[...])


def _tc_mid_body(h, p, g, degp, b, Wn, h_out, g_out):
    dinv = _dinv_of(degp[...])
    sfull = (p[0] + p[1] + g[...]) * dinv
    h_new = h[...] + jnp.maximum(sfull + b[...], 0.0)
    h_out[...] = h_new
    g_out[...] = jnp.dot(h_new, Wn[...], preferred_element_type=jnp.float32) \
        * dinv


def _tc_out_body(h, p, g, degp, b, ln_g, ln_b, oW1, ob1, oW2, ob2, y_out):
    dinv = _dinv_of(degp[...])
    sfull = (p[0] + p[1] + g[...]) * dinv
    hh = h[...] + jnp.maximum(sfull + b[...], 0.0)
    mu = jnp.mean(hh, axis=-1, keepdims=True)
    var = jnp.mean((hh - mu) ** 2, axis=-1, keepdims=True)
    hh = (hh - mu) / jnp.sqrt(var + 1e-5) * ln_g[...] + ln_b[...]
    hh = jnp.maximum(
        jnp.dot(hh, oW1[...], preferred_element_type=jnp.float32) + ob1[...],
        0.0)
    y_out[...] = jnp.dot(hh, oW2[...], preferred_element_type=jnp.float32) \
        + ob2[...]


_row_spec = pl.BlockSpec((_BN, _D), lambda i: (i, 0))
_p_spec = pl.BlockSpec((_NC, _BN, _D), lambda i: (0, i, 0))
_degp_spec = pl.BlockSpec((_NC, _BN, _DW), lambda i: (0, i, 0))
_w_spec = pl.BlockSpec((_D, _D), lambda i: (0, 0))
_b_spec = pl.BlockSpec((1, _D), lambda i: (0, 0))

_f32 = jnp.float32
_GRID = (_N // _BN,)

_tc_in = pl.pallas_call(
    _tc_in_body,
    grid=_GRID,
    in_specs=[_row_spec, _degp_spec, _w_spec, _b_spec, _w_spec],
    out_specs=[_row_spec, _row_spec],
    out_shape=[jax.ShapeDtypeStruct((_N, _D), _f32)] * 2,
)

_tc_mid = pl.pallas_call(
    _tc_mid_body,
    grid=_GRID,
    in_specs=[_row_spec, _p_spec, _row_spec, _degp_spec, _b_spec, _w_spec],
    out_specs=[_row_spec, _row_spec],
    out_shape=[jax.ShapeDtypeStruct((_N, _D), _f32)] * 2,
)

_tc_out = pl.pallas_call(
    _tc_out_body,
    grid=_GRID,
    in_specs=[_row_spec, _p_spec, _row_spec, _degp_spec, _b_spec,
              _b_spec, _b_spec, _w_spec, _b_spec,
              pl.BlockSpec((_D, 1), lambda i: (0, 0)),
              pl.BlockSpec((1, 1), lambda i: (0, 0))],
    out_specs=pl.BlockSpec((_BN, 1), lambda i: (i, 0)),
    out_shape=jax.ShapeDtypeStruct((_N, 1), _f32),
)


def kernel(x, edge_index, in_W, in_b, W0, b0, W1, b1, W2, b2, W3, b3,
           ln_g, ln_b, oW1, ob1, oW2, ob2):
    src3 = edge_index[0].astype(jnp.int32).reshape(_NW, _NCH, _CH)
    dst3 = edge_index[1].astype(jnp.int32).reshape(_NW, _NCH, _CH)
    ones_rows = jnp.ones((_CH, _DW), _f32)
    zrow = jnp.zeros((_RPT, _DW), _f32)
    zeros_slab = jnp.zeros((_RPT, _D), _f32)

    degp = _sc_degree(dst3, ones_rows, zrow)

    in_b2 = in_b.reshape(1, _D)
    h, g = _tc_in(x, degp, in_W, in_b2, W0)
    for b, Wn in ((b0, W1), (b1, W2), (b2, W3)):
        p = _sc_scatter(g, src3, dst3, zeros_slab)
        h, g = _tc_mid(h, p, g, degp, b.reshape(1, _D), Wn)
    p = _sc_scatter(g, src3, dst3, zeros_slab)
    y = _tc_out(h, p, g, degp, b3.reshape(1, _D), ln_g.reshape(1, _D),
                ln_b.reshape(1, _D), oW1, ob1.reshape(1, _D),
                oW2, ob2.reshape(1, 1))
    return y[:, 0]


# speedup vs baseline: 18.0643x; 1.2738x over previous
import functools

import jax
import jax.numpy as jnp
from jax import lax
from jax.experimental import pallas as pl
from jax.experimental.pallas import tpu as pltpu
from jax.experimental.pallas import tpu_sc as plsc

_N = 10000
_D = 128
_E = 320000
_NC = 2
_NS = 16
_NW = _NC * _NS
_EPT = _E // _NW
_CH = 80
_NCH = _EPT // _CH
_NPAD = 10240
_RPT = _NPAD // _NS
_NSC = 5
_SCH = 25
_DW = 128

_mesh = plsc.VectorSubcoreMesh(core_axis_name="c", subcore_axis_name="s")


@functools.partial(
    pl.kernel,
    out_type=jax.ShapeDtypeStruct((_NC, _NPAD, _DW), jnp.float32),
    mesh=_mesh,
    scratch_types=[
        pltpu.VMEM((_NCH, _CH), jnp.int32),
        pltpu.VMEM((_CH, _DW), jnp.float32),
        pltpu.VMEM_SHARED((_NPAD, _DW), jnp.float32),
    ],
)
def _sc_degree(dst_hbm, ones_hbm, zrow_hbm, out_hbm, dst_v, ones_v, acc):
    c = lax.axis_index("c")
    s = lax.axis_index("s")
    wid = c * _NS + s
    pltpu.sync_copy(zrow_hbm, acc.at[pl.ds(s * _RPT, _RPT)])
    pltpu.sync_copy(ones_hbm, ones_v)
    pltpu.sync_copy(dst_hbm.at[wid], dst_v)
    plsc.subcore_barrier()

    def body(j, carry):
        pltpu.sync_copy(ones_v, acc.at[dst_v.at[j]], add=True)
        return carry

    lax.fori_loop(0, _NCH, body, 0)
    plsc.subcore_barrier()
    pltpu.sync_copy(acc.at[pl.ds(s * _RPT, _RPT)],
                    out_hbm.at[c, pl.ds(s * _RPT, _RPT)])


@functools.partial(
    pl.kernel,
    out_type=jax.ShapeDtypeStruct((_NC, _NPAD, _D), jnp.float32),
    mesh=_mesh,
    scratch_types=[
        pltpu.VMEM((_SCH, _CH), jnp.int32),
        pltpu.VMEM((_SCH, _CH), jnp.int32),
        pltpu.VMEM((_CH, _D), jnp.float32),
        pltpu.VMEM((_CH, _D), jnp.float32),
        pltpu.VMEM_SHARED((_NPAD, _D), jnp.float32),
        pltpu.SemaphoreType.DMA,
    ],
)
def _sc_scatter(g_hbm, src_hbm, dst_hbm, zeros_hbm, out_hbm,
                src_v, dst_v, buf0, buf1, acc, sem):
    c = lax.axis_index("c")
    s = lax.axis_index("s")
    wid = c * _NS + s
    pltpu.sync_copy(zeros_hbm, acc.at[pl.ds(s * _RPT, _RPT)])
    plsc.subcore_barrier()

    def outer(o, carry):
        pltpu.sync_copy(src_hbm.at[wid, o], src_v)
        pltpu.sync_copy(dst_hbm.at[wid, o], dst_v)

        def body(i, carry2):
            j0 = 2 * i
            pltpu.async_copy(g_hbm.at[src_v.at[j0]], buf0, sem)
            pltpu.async_copy(g_hbm.at[src_v.at[j0 + 1]], buf1, sem)
            pltpu.make_async_copy(g_hbm.at[src_v.at[j0]], buf0, sem).wait()
            pltpu.sync_copy(buf0, acc.at[dst_v.at[j0]], add=True)
            pltpu.make_async_copy(g_hbm.at[src_v.at[j0 + 1]], buf1, sem).wait()
            pltpu.sync_copy(buf1, acc.at[dst_v.at[j0 + 1]], add=True)
            return carry2

        lax.fori_loop(0, _SCH // 2, body, 0)
        pltpu.async_copy(g_hbm.at[src_v.at[_SCH - 1]], buf0, sem).wait()
        pltpu.sync_copy(buf0, acc.at[dst_v.at[_SCH - 1]], add=True)
        return carry

    lax.fori_loop(0, _NSC, outer, 0)
    plsc.subcore_barrier()
    pltpu.sync_copy(acc.at[pl.ds(s * _RPT, _RPT)],
                    out_hbm.at[c, pl.ds(s * _RPT, _RPT)])


_BN = 2000


def _dinv_of(degp):
    deg = degp[0, :, 0] + degp[1, :, 0] + 1.0
    return lax.rsqrt(deg)[:, None]


def _tc_in_body(x, degp, in_W, in_b, W0, h_out, g_out):
    h = jnp.maximum(
        jnp.dot(x[...], in_W[...], preferred_element_type=jnp.float32)
        + in_b[...], 0.0)
    h_out[...] = h
    g_out[...] = jnp.dot(h, W0[...], preferred_element_type=jnp.float32) \
        * _dinv_of(degp[...])


def _tc_mid_body(h, p, g, degp, b, Wn, h_out, g_out):
    dinv = _dinv_of(degp[...])
    sfull = (p[0] + p[1] + g[...]) * dinv
    h_new = h[...] + jnp.maximum(sfull + b[...], 0.0)
    h_out[...] = h_new
    g_out[...] = jnp.dot(h_new, Wn[...], preferred_element_type=jnp.float32) \
        * dinv


def _tc_out_body(h, p, g, degp, b, ln_g, ln_b, oW1, ob1, oW2, ob2, y_out):
    dinv = _dinv_of(degp[...])
    sfull = (p[0] + p[1] + g[...]) * dinv
    hh = h[...] + jnp.maximum(sfull + b[...], 0.0)
    mu = jnp.mean(hh, axis=-1, keepdims=True)
    var = jnp.mean((hh - mu) ** 2, axis=-1, keepdims=True)
    hh = (hh - mu) / jnp.sqrt(var + 1e-5) * ln_g[...] + ln_b[...]
    hh = jnp.maximum(
        jnp.dot(hh, oW1[...], preferred_element_type=jnp.float32) + ob1[...],
        0.0)
    y_out[...] = jnp.dot(hh, oW2[...], preferred_element_type=jnp.float32) \
        + ob2[...]


_row_spec = pl.BlockSpec((_BN, _D), lambda i: (i, 0))
_p_spec = pl.BlockSpec((_NC, _BN, _D), lambda i: (0, i, 0))
_degp_spec = pl.BlockSpec((_NC, _BN, _DW), lambda i: (0, i, 0))
_w_spec = pl.BlockSpec((_D, _D), lambda i: (0, 0))
_b_spec = pl.BlockSpec((1, _D), lambda i: (0, 0))

_f32 = jnp.float32
_GRID = (_N // _BN,)

_tc_in = pl.pallas_call(
    _tc_in_body,
    grid=_GRID,
    in_specs=[_row_spec, _degp_spec, _w_spec, _b_spec, _w_spec],
    out_specs=[_row_spec, _row_spec],
    out_shape=[jax.ShapeDtypeStruct((_N, _D), _f32)] * 2,
)

_tc_mid = pl.pallas_call(
    _tc_mid_body,
    grid=_GRID,
    in_specs=[_row_spec, _p_spec, _row_spec, _degp_spec, _b_spec, _w_spec],
    out_specs=[_row_spec, _row_spec],
    out_shape=[jax.ShapeDtypeStruct((_N, _D), _f32)] * 2,
)

_tc_out = pl.pallas_call(
    _tc_out_body,
    grid=_GRID,
    in_specs=[_row_spec, _p_spec, _row_spec, _degp_spec, _b_spec,
              _b_spec, _b_spec, _w_spec, _b_spec,
              pl.BlockSpec((_D, 1), lambda i: (0, 0)),
              pl.BlockSpec((1, 1), lambda i: (0, 0))],
    out_specs=pl.BlockSpec((_BN, 1), lambda i: (i, 0)),
    out_shape=jax.ShapeDtypeStruct((_N, 1), _f32),
)


def kernel(x, edge_index, in_W, in_b, W0, b0, W1, b1, W2, b2, W3, b3,
           ln_g, ln_b, oW1, ob1, oW2, ob2):
    src3 = edge_index[0].astype(jnp.int32).reshape(_NW, _NCH, _CH)
    dst3 = edge_index[1].astype(jnp.int32).reshape(_NW, _NCH, _CH)
    src4 = src3.reshape(_NW, _NSC, _SCH, _CH)
    dst4 = dst3.reshape(_NW, _NSC, _SCH, _CH)
    ones_rows = jnp.ones((_CH, _DW), _f32)
    zrow = jnp.zeros((_RPT, _DW), _f32)
    zeros_slab = jnp.zeros((_RPT, _D), _f32)

    degp = _sc_degree(dst3, ones_rows, zrow)

    in_b2 = in_b.reshape(1, _D)
    h, g = _tc_in(x, degp, in_W, in_b2, W0)
    for b, Wn in ((b0, W1), (b1, W2), (b2, W3)):
        p = _sc_scatter(g, src4, dst4, zeros_slab)
        h, g = _tc_mid(h, p, g, degp, b.reshape(1, _D), Wn)
    p = _sc_scatter(g, src4, dst4, zeros_slab)
    y = _tc_out(h, p, g, degp, b3.reshape(1, _D), ln_g.reshape(1, _D),
                ln_b.reshape(1, _D), oW1, ob1.reshape(1, _D),
                oW2, ob2.reshape(1, 1))
    return y[:, 0]


# speedup vs baseline: 19.6406x; 1.0873x over previous
import functools

import jax
import jax.numpy as jnp
from jax import lax
from jax.experimental import pallas as pl
from jax.experimental.pallas import tpu as pltpu
from jax.experimental.pallas import tpu_sc as plsc

_N = 10000
_D = 128
_E = 320000
_NC = 2
_NS = 16
_NW = _NC * _NS
_EPT = _E // _NW
_CH = 80
_NCH = _EPT // _CH
_NPAD = 10240
_RPT = _NPAD // _NS
_NSC = 5
_SCH = 25
_DW = 128

_mesh = plsc.VectorSubcoreMesh(core_axis_name="c", subcore_axis_name="s")


@functools.partial(
    pl.kernel,
    out_type=jax.ShapeDtypeStruct((_NC, _NPAD, _DW), jnp.float32),
    mesh=_mesh,
    scratch_types=[
        pltpu.VMEM((_NCH, _CH), jnp.int32),
        pltpu.VMEM((_CH, _DW), jnp.float32),
        pltpu.VMEM_SHARED((_NPAD, _DW), jnp.float32),
    ],
)
def _sc_degree(dst_hbm, ones_hbm, zrow_hbm, out_hbm, dst_v, ones_v, acc):
    c = lax.axis_index("c")
    s = lax.axis_index("s")
    wid = c * _NS + s
    pltpu.sync_copy(zrow_hbm, acc.at[pl.ds(s * _RPT, _RPT)])
    pltpu.sync_copy(ones_hbm, ones_v)
    pltpu.sync_copy(dst_hbm.at[wid], dst_v)
    plsc.subcore_barrier()

    def body(j, carry):
        pltpu.sync_copy(ones_v, acc.at[dst_v.at[j]], add=True)
        return carry

    lax.fori_loop(0, _NCH, body, 0)
    plsc.subcore_barrier()
    pltpu.sync_copy(acc.at[pl.ds(s * _RPT, _RPT)],
                    out_hbm.at[c, pl.ds(s * _RPT, _RPT)])


@functools.partial(
    pl.kernel,
    out_type=jax.ShapeDtypeStruct((_NC, _NPAD, _D), jnp.float32),
    mesh=_mesh,
    scratch_types=[
        pltpu.VMEM((_SCH, _CH), jnp.int32),
        pltpu.VMEM((_SCH, _CH), jnp.int32),
        pltpu.VMEM((_CH, _D), jnp.float32),
        pltpu.VMEM((_CH, _D), jnp.float32),
        pltpu.VMEM((_CH, _D), jnp.float32),
        pltpu.VMEM_SHARED((_NPAD, _D), jnp.float32),
        pltpu.SemaphoreType.DMA,
    ],
)
def _sc_scatter(g_hbm, src_hbm, dst_hbm, zeros_hbm, out_hbm,
                src_v, dst_v, buf0, buf1, buf2, acc, sem):
    c = lax.axis_index("c")
    s = lax.axis_index("s")
    wid = c * _NS + s
    pltpu.sync_copy(zeros_hbm, acc.at[pl.ds(s * _RPT, _RPT)])
    plsc.subcore_barrier()

    def outer(o, carry):
        pltpu.sync_copy(src_hbm.at[wid, o], src_v)
        pltpu.sync_copy(dst_hbm.at[wid, o], dst_v)

        def body(i, carry2):
            j0 = 3 * i
            pltpu.async_copy(g_hbm.at[src_v.at[j0]], buf0, sem)
            pltpu.async_copy(g_hbm.at[src_v.at[j0 + 1]], buf1, sem)
            pltpu.async_copy(g_hbm.at[src_v.at[j0 + 2]], buf2, sem)
            pltpu.make_async_copy(g_hbm.at[src_v.at[j0]], buf0, sem).wait()
            pltpu.sync_copy(buf0, acc.at[dst_v.at[j0]], add=True)
            pltpu.make_async_copy(g_hbm.at[src_v.at[j0 + 1]], buf1, sem).wait()
            pltpu.sync_copy(buf1, acc.at[dst_v.at[j0 + 1]], add=True)
            pltpu.make_async_copy(g_hbm.at[src_v.at[j0 + 2]], buf2, sem).wait()
            pltpu.sync_copy(buf2, acc.at[dst_v.at[j0 + 2]], add=True)
            return carry2

        lax.fori_loop(0, _SCH // 3, body, 0)
        pltpu.async_copy(g_hbm.at[src_v.at[_SCH - 1]], buf0, sem).wait()
        pltpu.sync_copy(buf0, acc.at[dst_v.at[_SCH - 1]], add=True)
        return carry

    lax.fori_loop(0, _NSC, outer, 0)
    plsc.subcore_barrier()
    pltpu.sync_copy(acc.at[pl.ds(s * _RPT, _RPT)],
                    out_hbm.at[c, pl.ds(s * _RPT, _RPT)])


_BN = 2000


def _dinv_of(degp):
    deg = degp[0, :, 0] + degp[1, :, 0] + 1.0
    return lax.rsqrt(deg)[:, None]


def _tc_in_body(x, degp, in_W, in_b, W0, h_out, g_out):
    h = jnp.maximum(
        jnp.dot(x[...], in_W[...], preferred_element_type=jnp.float32)
        + in_b[...], 0.0)
    h_out[...] = h
    g_out[...] = jnp.dot(h, W0[...], preferred_element_type=jnp.float32) \
        * _dinv_of(degp[...])


def _tc_mid_body(h, p, g, degp, b, Wn, h_out, g_out):
    dinv = _dinv_of(degp[...])
    sfull = (p[0] + p[1] + g[...]) * dinv
    h_new = h[...] + jnp.maximum(sfull + b[...], 0.0)
    h_out[...] = h_new
    g_out[...] = jnp.dot(h_new, Wn[...], preferred_element_type=jnp.float32) \
        * dinv


def _tc_out_body(h, p, g, degp, b, ln_g, ln_b, oW1, ob1, oW2, ob2, y_out):
    dinv = _dinv_of(degp[...])
    sfull = (p[0] + p[1] + g[...]) * dinv
    hh = h[...] + jnp.maximum(sfull + b[...], 0.0)
    mu = jnp.mean(hh, axis=-1, keepdims=True)
    var = jnp.mean((hh - mu) ** 2, axis=-1, keepdims=True)
    hh = (hh - mu) / jnp.sqrt(var + 1e-5) * ln_g[...] + ln_b[...]
    hh = jnp.maximum(
        jnp.dot(hh, oW1[...], preferred_element_type=jnp.float32) + ob1[...],
        0.0)
    y_out[...] = jnp.dot(hh, oW2[...], preferred_element_type=jnp.float32) \
        + ob2[...]


_row_spec = pl.BlockSpec((_BN, _D), lambda i: (i, 0))
_p_spec = pl.BlockSpec((_NC, _BN, _D), lambda i: (0, i, 0))
_degp_spec = pl.BlockSpec((_NC, _BN, _DW), lambda i: (0, i, 0))
_w_spec = pl.BlockSpec((_D, _D), lambda i: (0, 0))
_b_spec = pl.BlockSpec((1, _D), lambda i: (0, 0))

_f32 = jnp.float32
_GRID = (_N // _BN,)

_tc_in = pl.pallas_call(
    _tc_in_body,
    grid=_GRID,
    in_specs=[_row_spec, _degp_spec, _w_spec, _b_spec, _w_spec],
    out_specs=[_row_spec, _row_spec],
    out_shape=[jax.ShapeDtypeStruct((_N, _D), _f32)] * 2,
)

_tc_mid = pl.pallas_call(
    _tc_mid_body,
    grid=_GRID,
    in_specs=[_row_spec, _p_spec, _row_spec, _degp_spec, _b_spec, _w_spec],
    out_specs=[_row_spec, _row_spec],
    out_shape=[jax.ShapeDtypeStruct((_N, _D), _f32)] * 2,
)

_tc_out = pl.pallas_call(
    _tc_out_body,
    grid=_GRID,
    in_specs=[_row_spec, _p_spec, _row_spec, _degp_spec, _b_spec,
              _b_spec, _b_spec, _w_spec, _b_spec,
              pl.BlockSpec((_D, 1), lambda i: (0, 0)),
              pl.BlockSpec((1, 1), lambda i: (0, 0))],
    out_specs=pl.BlockSpec((_BN, 1), lambda i: (i, 0)),
    out_shape=jax.ShapeDtypeStruct((_N, 1), _f32),
)


def kernel(x, edge_index, in_W, in_b, W0, b0, W1, b1, W2, b2, W3, b3,
           ln_g, ln_b, oW1, ob1, oW2, ob2):
    src3 = edge_index[0].astype(jnp.int32).reshape(_NW, _NCH, _CH)
    dst3 = edge_index[1].astype(jnp.int32).reshape(_NW, _NCH, _CH)
    src4 = src3.reshape(_NW, _NSC, _SCH, _CH)
    dst4 = dst3.reshape(_NW, _NSC, _SCH, _CH)
    ones_rows = jnp.ones((_CH, _DW), _f32)
    zrow = jnp.zeros((_RPT, _DW), _f32)
    zeros_slab = jnp.zeros((_RPT, _D), _f32)

    degp = _sc_degree(dst3, ones_rows, zrow)

    in_b2 = in_b.reshape(1, _D)
    h, g = _tc_in(x, degp, in_W, in_b2, W0)
    for b, Wn in ((b0, W1), (b1, W2), (b2, W3)):
        p = _sc_scatter(g, src4, dst4, zeros_slab)
        h, g = _tc_mid(h, p, g, degp, b.reshape(1, _D), Wn)
    p = _sc_scatter(g, src4, dst4, zeros_slab)
    y = _tc_out(h, p, g, degp, b3.reshape(1, _D), ln_g.reshape(1, _D),
                ln_b.reshape(1, _D), oW1, ob1.reshape(1, _D),
                oW2, ob2.reshape(1, 1))
    return y[:, 0]


# speedup vs baseline: 20.3293x; 1.0351x over previous
import functools

import jax
import jax.numpy as jnp
from jax import lax
from jax.experimental import pallas as pl
from jax.experimental.pallas import tpu as pltpu
from jax.experimental.pallas import tpu_sc as plsc

_N = 10000
_D = 128
_E = 320000
_NC = 2
_NS = 16
_NW = _NC * _NS
_EPT = _E // _NW
_CH = 80
_NCH = _EPT // _CH
_NPAD = 10240
_RPT = _NPAD // _NS
_NSC = 5
_SCH = 25
_DW = 128

_mesh = plsc.VectorSubcoreMesh(core_axis_name="c", subcore_axis_name="s")


@functools.partial(
    pl.kernel,
    out_type=jax.ShapeDtypeStruct((_NC, _NPAD, _DW), jnp.float32),
    mesh=_mesh,
    scratch_types=[
        pltpu.VMEM((_NCH, _CH), jnp.int32),
        pltpu.VMEM((_CH, _DW), jnp.float32),
        pltpu.VMEM_SHARED((_NPAD, _DW), jnp.float32),
    ],
)
def _sc_degree(dst_hbm, ones_hbm, zrow_hbm, out_hbm, dst_v, ones_v, acc):
    c = lax.axis_index("c")
    s = lax.axis_index("s")
    wid = c * _NS + s
    pltpu.sync_copy(zrow_hbm, acc.at[pl.ds(s * _RPT, _RPT)])
    pltpu.sync_copy(ones_hbm, ones_v)
    pltpu.sync_copy(dst_hbm.at[wid], dst_v)
    plsc.subcore_barrier()

    def body(j, carry):
        pltpu.sync_copy(ones_v, acc.at[dst_v.at[j]], add=True)
        return carry

    lax.fori_loop(0, _NCH, body, 0)
    plsc.subcore_barrier()
    pltpu.sync_copy(acc.at[pl.ds(s * _RPT, _RPT)],
                    out_hbm.at[c, pl.ds(s * _RPT, _RPT)])


@functools.partial(
    pl.kernel,
    out_type=jax.ShapeDtypeStruct((_NC, _NPAD, _D), jnp.float32),
    mesh=_mesh,
    scratch_types=[
        pltpu.VMEM((_SCH, _CH), jnp.int32),
        pltpu.VMEM((_SCH, _CH), jnp.int32),
        pltpu.VMEM((_CH, _D), jnp.float32),
        pltpu.VMEM((_CH, _D), jnp.float32),
        pltpu.VMEM((_CH, _D), jnp.float32),
        pltpu.VMEM((_CH, _D), jnp.float32),
        pltpu.VMEM_SHARED((_NPAD, _D), jnp.float32),
        pltpu.SemaphoreType.DMA,
    ],
)
def _sc_scatter(g_hbm, src_hbm, dst_hbm, zeros_hbm, out_hbm,
                src_v, dst_v, buf0, buf1, buf2, buf3, acc, sem):
    c = lax.axis_index("c")
    s = lax.axis_index("s")
    wid = c * _NS + s
    pltpu.sync_copy(zeros_hbm, acc.at[pl.ds(s * _RPT, _RPT)])
    plsc.subcore_barrier()

    def outer(o, carry):
        pltpu.sync_copy(src_hbm.at[wid, o], src_v)
        pltpu.sync_copy(dst_hbm.at[wid, o], dst_v)

        def body(i, carry2):
            j0 = 4 * i
            pltpu.async_copy(g_hbm.at[src_v.at[j0]], buf0, sem)
            pltpu.async_copy(g_hbm.at[src_v.at[j0 + 1]], buf1, sem)
            pltpu.async_copy(g_hbm.at[src_v.at[j0 + 2]], buf2, sem)
            pltpu.async_copy(g_hbm.at[src_v.at[j0 + 3]], buf3, sem)
            pltpu.make_async_copy(g_hbm.at[src_v.at[j0]], buf0, sem).wait()
            pltpu.sync_copy(buf0, acc.at[dst_v.at[j0]], add=True)
            pltpu.make_async_copy(g_hbm.at[src_v.at[j0 + 1]], buf1, sem).wait()
            pltpu.sync_copy(buf1, acc.at[dst_v.at[j0 + 1]], add=True)
            pltpu.make_async_copy(g_hbm.at[src_v.at[j0 + 2]], buf2, sem).wait()
            pltpu.sync_copy(buf2, acc.at[dst_v.at[j0 + 2]], add=True)
            pltpu.make_async_copy(g_hbm.at[src_v.at[j0 + 3]], buf3, sem).wait()
            pltpu.sync_copy(buf3, acc.at[dst_v.at[j0 + 3]], add=True)
            return carry2

        lax.fori_loop(0, _SCH // 4, body, 0)
        pltpu.async_copy(g_hbm.at[src_v.at[_SCH - 1]], buf0, sem).wait()
        pltpu.sync_copy(buf0, acc.at[dst_v.at[_SCH - 1]], add=True)
        return carry

    lax.fori_loop(0, _NSC, outer, 0)
    plsc.subcore_barrier()
    pltpu.sync_copy(acc.at[pl.ds(s * _RPT, _RPT)],
                    out_hbm.at[c, pl.ds(s * _RPT, _RPT)])


_BN = 2000


def _dinv_of(degp):
    deg = degp[0, :, 0] + degp[1, :, 0] + 1.0
    return lax.rsqrt(deg)[:, None]


def _tc_in_body(x, degp, in_W, in_b, W0, h_out, g_out):
    h = jnp.maximum(
        jnp.dot(x[...], in_W[...], preferred_element_type=jnp.float32)
        + in_b[...], 0.0)
    h_out[...] = h
    g_out[...] = jnp.dot(h, W0[...], preferred_element_type=jnp.float32) \
        * _dinv_of(degp[...])


def _tc_mid_body(h, p, g, degp, b, Wn, h_out, g_out):
    dinv = _dinv_of(degp[...])
    sfull = (p[0] + p[1] + g[...]) * dinv
    h_new = h[...] + jnp.maximum(sfull + b[...], 0.0)
    h_out[...] = h_new
    g_out[...] = jnp.dot(h_new, Wn[...], preferred_element_type=jnp.float32) \
        * dinv


def _tc_out_body(h, p, g, degp, b, ln_g, ln_b, oW1, ob1, oW2, ob2, y_out):
    dinv = _dinv_of(degp[...])
    sfull = (p[0] + p[1] + g[...]) * dinv
    hh = h[...] + jnp.maximum(sfull + b[...], 0.0)
    mu = jnp.mean(hh, axis=-1, keepdims=True)
    var = jnp.mean((hh - mu) ** 2, axis=-1, keepdims=True)
    hh = (hh - mu) / jnp.sqrt(var + 1e-5) * ln_g[...] + ln_b[...]
    hh = jnp.maximum(
        jnp.dot(hh, oW1[...], preferred_element_type=jnp.float32) + ob1[...],
        0.0)
    y_out[...] = jnp.dot(hh, oW2[...], preferred_element_type=jnp.float32) \
        + ob2[...]


_row_spec = pl.BlockSpec((_BN, _D), lambda i: (i, 0))
_p_spec = pl.BlockSpec((_NC, _BN, _D), lambda i: (0, i, 0))
_degp_spec = pl.BlockSpec((_NC, _BN, _DW), lambda i: (0, i, 0))
_w_spec = pl.BlockSpec((_D, _D), lambda i: (0, 0))
_b_spec = pl.BlockSpec((1, _D), lambda i: (0, 0))

_f32 = jnp.float32
_GRID = (_N // _BN,)

_tc_in = pl.pallas_call(
    _tc_in_body,
    grid=_GRID,
    in_specs=[_row_spec, _degp_spec, _w_spec, _b_spec, _w_spec],
    out_specs=[_row_spec, _row_spec],
    out_shape=[jax.ShapeDtypeStruct((_N, _D), _f32)] * 2,
)

_tc_mid = pl.pallas_call(
    _tc_mid_body,
    grid=_GRID,
    in_specs=[_row_spec, _p_spec, _row_spec, _degp_spec, _b_spec, _w_spec],
    out_specs=[_row_spec, _row_spec],
    out_shape=[jax.ShapeDtypeStruct((_N, _D), _f32)] * 2,
)

_tc_out = pl.pallas_call(
    _tc_out_body,
    grid=_GRID,
    in_specs=[_row_spec, _p_spec, _row_spec, _degp_spec, _b_spec,
              _b_spec, _b_spec, _w_spec, _b_spec,
              pl.BlockSpec((_D, 1), lambda i: (0, 0)),
              pl.BlockSpec((1, 1), lambda i: (0, 0))],
    out_specs=pl.BlockSpec((_BN, 1), lambda i: (i, 0)),
    out_shape=jax.ShapeDtypeStruct((_N, 1), _f32),
)


def kernel(x, edge_index, in_W, in_b, W0, b0, W1, b1, W2, b2, W3, b3,
           ln_g, ln_b, oW1, ob1, oW2, ob2):
    src3 = edge_index[0].astype(jnp.int32).reshape(_NW, _NCH, _CH)
    dst3 = edge_index[1].astype(jnp.int32).reshape(_NW, _NCH, _CH)
    src4 = src3.reshape(_NW, _NSC, _SCH, _CH)
    dst4 = dst3.reshape(_NW, _NSC, _SCH, _CH)
    ones_rows = jnp.ones((_CH, _DW), _f32)
    zrow = jnp.zeros((_RPT, _DW), _f32)
    zeros_slab = jnp.zeros((_RPT, _D), _f32)

    degp = _sc_degree(dst3, ones_rows, zrow)

    in_b2 = in_b.reshape(1, _D)
    h, g = _tc_in(x, degp, in_W, in_b2, W0)
    for b, Wn in ((b0, W1), (b1, W2), (b2, W3)):
        p = _sc_scatter(g, src4, dst4, zeros_slab)
        h, g = _tc_mid(h, p, g, degp, b.reshape(1, _D), Wn)
    p = _sc_scatter(g, src4, dst4, zeros_slab)
    y = _tc_out(h, p, g, degp, b3.reshape(1, _D), ln_g.reshape(1, _D),
                ln_b.reshape(1, _D), oW1, ob1.reshape(1, _D),
                oW2, ob2.reshape(1, 1))
    return y[:, 0]


# speedup vs baseline: 20.4242x; 1.0047x over previous
import functools

import jax
import jax.numpy as jnp
from jax import lax
from jax.experimental import pallas as pl
from jax.experimental.pallas import tpu as pltpu
from jax.experimental.pallas import tpu_sc as plsc

_N = 10000
_D = 128
_E = 320000
_NC = 2
_NS = 16
_NW = _NC * _NS
_EPT = _E // _NW
_CH = 80
_NCH = _EPT // _CH
_NPAD = 10240
_RPT = _NPAD // _NS
_NSC = 5
_SCH = 25
_DW = 128

_mesh = plsc.VectorSubcoreMesh(core_axis_name="c", subcore_axis_name="s")


@functools.partial(
    pl.kernel,
    out_type=jax.ShapeDtypeStruct((_NC, _NPAD, _DW), jnp.float32),
    mesh=_mesh,
    scratch_types=[
        pltpu.VMEM((_NCH, _CH), jnp.int32),
        pltpu.VMEM((_CH, _DW), jnp.float32),
        pltpu.VMEM_SHARED((_NPAD, _DW), jnp.float32),
    ],
)
def _sc_degree(dst_hbm, ones_hbm, zrow_hbm, out_hbm, dst_v, ones_v, acc):
    c = lax.axis_index("c")
    s = lax.axis_index("s")
    wid = c * _NS + s
    pltpu.sync_copy(zrow_hbm, acc.at[pl.ds(s * _RPT, _RPT)])
    pltpu.sync_copy(ones_hbm, ones_v)
    pltpu.sync_copy(dst_hbm.at[wid], dst_v)
    plsc.subcore_barrier()

    def body(j, carry):
        pltpu.sync_copy(ones_v, acc.at[dst_v.at[j]], add=True)
        return carry

    lax.fori_loop(0, _NCH, body, 0)
    plsc.subcore_barrier()
    pltpu.sync_copy(acc.at[pl.ds(s * _RPT, _RPT)],
                    out_hbm.at[c, pl.ds(s * _RPT, _RPT)])


@functools.partial(
    pl.kernel,
    out_type=jax.ShapeDtypeStruct((_NC, _NPAD, _D), jnp.float32),
    mesh=_mesh,
    scratch_types=[
        pltpu.VMEM((_SCH, _CH), jnp.int32),
        pltpu.VMEM((_SCH, _CH), jnp.int32),
        pltpu.VMEM((_CH, _D), jnp.float32),
        pltpu.VMEM((_CH, _D), jnp.float32),
        pltpu.VMEM((_CH, _D), jnp.float32),
        pltpu.VMEM((_CH, _D), jnp.float32),
        pltpu.VMEM_SHARED((_NPAD, _D), jnp.float32),
        pltpu.SemaphoreType.DMA,
    ],
)
def _sc_scatter(g_hbm, src_hbm, dst_hbm, zeros_hbm, out_hbm,
                src_v, dst_v, buf0, buf1, buf2, buf3, acc, sem):
    c = lax.axis_index("c")
    s = lax.axis_index("s")
    wid = c * _NS + s
    pltpu.sync_copy(zeros_hbm, acc.at[pl.ds(s * _RPT, _RPT)])
    plsc.subcore_barrier()

    def outer(o, carry):
        pltpu.sync_copy(src_hbm.at[wid, o], src_v)
        pltpu.sync_copy(dst_hbm.at[wid, o], dst_v)

        def body(i, carry2):
            j0 = 4 * i
            pltpu.async_copy(g_hbm.at[src_v.at[j0]], buf0, sem)
            pltpu.async_copy(g_hbm.at[src_v.at[j0 + 1]], buf1, sem)
            pltpu.async_copy(g_hbm.at[src_v.at[j0 + 2]], buf2, sem)
            pltpu.async_copy(g_hbm.at[src_v.at[j0 + 3]], buf3, sem)
            pltpu.make_async_copy(g_hbm.at[src_v.at[j0]], buf0, sem).wait()
            pltpu.sync_copy(buf0, acc.at[dst_v.at[j0]], add=True)
            pltpu.make_async_copy(g_hbm.at[src_v.at[j0 + 1]], buf1, sem).wait()
            pltpu.sync_copy(buf1, acc.at[dst_v.at[j0 + 1]], add=True)
            pltpu.make_async_copy(g_hbm.at[src_v.at[j0 + 2]], buf2, sem).wait()
            pltpu.sync_copy(buf2, acc.at[dst_v.at[j0 + 2]], add=True)
            pltpu.make_async_copy(g_hbm.at[src_v.at[j0 + 3]], buf3, sem).wait()
            pltpu.sync_copy(buf3, acc.at[dst_v.at[j0 + 3]], add=True)
            return carry2

        lax.fori_loop(0, _SCH // 4, body, 0)
        pltpu.async_copy(g_hbm.at[src_v.at[_SCH - 1]], buf0, sem).wait()
        pltpu.sync_copy(buf0, acc.at[dst_v.at[_SCH - 1]], add=True)
        return carry

    lax.fori_loop(0, _NSC, outer, 0)
    plsc.subcore_barrier()
    pltpu.sync_copy(acc.at[pl.ds(s * _RPT, _RPT)],
                    out_hbm.at[c, pl.ds(s * _RPT, _RPT)])


_BN = 2000


def _dinv_of(degp):
    deg = degp[0, :, 0] + degp[1, :, 0] + 1.0
    return lax.rsqrt(deg)[:, None]


def _tc_h0_body(x, in_W, in_b, h_out):
    h_out[...] = jnp.maximum(
        jnp.dot(x[...], in_W[...], preferred_element_type=jnp.float32)
        + in_b[...], 0.0)


def _tc_g0_body(h, degp, W0, g_out, dinv_out):
    dinv = _dinv_of(degp[...])
    dinv_out[...] = dinv
    g_out[...] = jnp.dot(h[...], W0[...], preferred_element_type=jnp.float32) \
        * dinv


def _tc_mid_body(h, p, g, dinv_in, b, Wn, h_out, g_out):
    dinv = dinv_in[...]
    sfull = (p[0] + p[1] + g[...]) * dinv
    h_new = h[...] + jnp.maximum(sfull + b[...], 0.0)
    h_out[...] = h_new
    g_out[...] = jnp.dot(h_new, Wn[...], preferred_element_type=jnp.float32) \
        * dinv


def _tc_out_body(h, p, g, dinv_in, b, ln_g, ln_b, oW1, ob1, oW2, ob2, y_out):
    dinv = dinv_in[...]
    sfull = (p[0] + p[1] + g[...]) * dinv
    hh = h[...] + jnp.maximum(sfull + b[...], 0.0)
    mu = jnp.mean(hh, axis=-1, keepdims=True)
    var = jnp.mean((hh - mu) ** 2, axis=-1, keepdims=True)
    hh = (hh - mu) / jnp.sqrt(var + 1e-5) * ln_g[...] + ln_b[...]
    hh = jnp.maximum(
        jnp.dot(hh, oW1[...], preferred_element_type=jnp.float32) + ob1[...],
        0.0)
    y_out[...] = jnp.dot(hh, oW2[...], preferred_element_type=jnp.float32) \
        + ob2[...]


_row_spec = pl.BlockSpec((_BN, _D), lambda i: (i, 0))
_p_spec = pl.BlockSpec((_NC, _BN, _D), lambda i: (0, i, 0))
_degp_spec = pl.BlockSpec((_NC, _BN, _DW), lambda i: (0, i, 0))
_w_spec = pl.BlockSpec((_D, _D), lambda i: (0, 0))
_b_spec = pl.BlockSpec((1, _D), lambda i: (0, 0))

_f32 = jnp.float32
_GRID = (_N // _BN,)

_dinv_spec = pl.BlockSpec((_BN, 1), lambda i: (i, 0))

_tc_h0 = pl.pallas_call(
    _tc_h0_body,
    grid=_GRID,
    in_specs=[_row_spec, _w_spec, _b_spec],
    out_specs=_row_spec,
    out_shape=jax.ShapeDtypeStruct((_N, _D), _f32),
)

_tc_g0 = pl.pallas_call(
    _tc_g0_body,
    grid=_GRID,
    in_specs=[_row_spec, _degp_spec, _w_spec],
    out_specs=[_row_spec, _dinv_spec],
    out_shape=[jax.ShapeDtypeStruct((_N, _D), _f32),
               jax.ShapeDtypeStruct((_N, 1), _f32)],
)

_tc_mid = pl.pallas_call(
    _tc_mid_body,
    grid=_GRID,
    in_specs=[_row_spec, _p_spec, _row_spec, _dinv_spec, _b_spec, _w_spec],
    out_specs=[_row_spec, _row_spec],
    out_shape=[jax.ShapeDtypeStruct((_N, _D), _f32)] * 2,
)

_tc_out = pl.pallas_call(
    _tc_out_body,
    grid=_GRID,
    in_specs=[_row_spec, _p_spec, _row_spec, _dinv_spec, _b_spec,
              _b_spec, _b_spec, _w_spec, _b_spec,
              pl.BlockSpec((_D, 1), lambda i: (0, 0)),
              pl.BlockSpec((1, 1), lambda i: (0, 0))],
    out_specs=pl.BlockSpec((_BN, 1), lambda i: (i, 0)),
    out_shape=jax.ShapeDtypeStruct((_N, 1), _f32),
)


def kernel(x, edge_index, in_W, in_b, W0, b0, W1, b1, W2, b2, W3, b3,
           ln_g, ln_b, oW1, ob1, oW2, ob2):
    src3 = edge_index[0].astype(jnp.int32).reshape(_NW, _NCH, _CH)
    dst3 = edge_index[1].astype(jnp.int32).reshape(_NW, _NCH, _CH)
    src4 = src3.reshape(_NW, _NSC, _SCH, _CH)
    dst4 = dst3.reshape(_NW, _NSC, _SCH, _CH)
    ones_rows = jnp.ones((_CH, _DW), _f32)
    zrow = jnp.zeros((_RPT, _DW), _f32)
    zeros_slab = jnp.zeros((_RPT, _D), _f32)

    degp = _sc_degree(dst3, ones_rows, zrow)
    h = _tc_h0(x, in_W, in_b.reshape(1, _D))
    g, dinv = _tc_g0(h, degp, W0)
    for b, Wn in ((b0, W1), (b1, W2), (b2, W3)):
        p = _sc_scatter(g, src4, dst4, zeros_slab)
        h, g = _tc_mid(h, p, g, dinv, b.reshape(1, _D), Wn)
    p = _sc_scatter(g, src4, dst4, zeros_slab)
    y = _tc_out(h, p, g, dinv, b3.reshape(1, _D), ln_g.reshape(1, _D),
                ln_b.reshape(1, _D), oW1, ob1.reshape(1, _D),
                oW2, ob2.reshape(1, 1))
    return y[:, 0]
